# baseline (speedup 1.0000x reference)
import functools

import jax
import jax.numpy as jnp
from jax import lax
from jax.experimental import pallas as pl
from jax.experimental.pallas import tpu as pltpu
from jax.experimental.pallas import tpu_sc as plsc

N = 10000
E = 320000
D = 128

NC = 2
NS = 16
NW = NC * NS

CHUNK = 128
CW = 80
EPAD = NW * CW * CHUNK
NPAD = 10112
WROWS = NPAD // NS

TCB = 10000

_mesh = plsc.VectorSubcoreMesh(core_axis_name="c", subcore_axis_name="s")


def _deg_body(dstp_hbm, ones_hbm, zeros_hbm, out_hbm, dst_v, ones_v, acc_sh,
              sem):
    c = lax.axis_index("c")
    s = lax.axis_index("s")
    w = s * NC + c
    pltpu.sync_copy(zeros_hbm.at[pl.ds(s * WROWS, WROWS)],
                    acc_sh.at[pl.ds(s * WROWS, WROWS)])
    pltpu.sync_copy(ones_hbm, ones_v)
    pltpu.sync_copy(dstp_hbm.at[pl.ds(w * CW, CW)], dst_v)
    plsc.subcore_barrier()

    def body(j, carry):
        pltpu.sync_copy(ones_v, acc_sh.at[dst_v.at[j]], add=True)
        return carry

    lax.fori_loop(0, CW, body, 0)
    plsc.subcore_barrier()
    pltpu.sync_copy(acc_sh.at[pl.ds(s * WROWS, WROWS)],
                    out_hbm.at[c, pl.ds(s * WROWS, WROWS)])


_deg_kernel = functools.partial(
    pl.kernel,
    out_type=jax.ShapeDtypeStruct((NC, NPAD, D), jnp.float32),
    mesh=_mesh,
    scratch_types=[
        pltpu.VMEM((CW, CHUNK), jnp.int32),
        pltpu.VMEM((CHUNK, D), jnp.float32),
        pltpu.VMEM_SHARED((NPAD, D), jnp.float32),
        pltpu.SemaphoreType.DMA,
    ],
)(_deg_body)


NBUF = 2


CW0 = 120
CW1 = 2 * CW - CW0
SLAB = 40


def _agg_body(g_hbm, srcp_hbm, dstp_hbm, zeros_hbm, out_hbm, src_v, dst_v,
              r0, r1, acc_sh, *sems):
    c = lax.axis_index("c")
    s = lax.axis_index("s")
    rows = [r0, r1]
    pltpu.sync_copy(zeros_hbm.at[pl.ds(s * WROWS, WROWS)],
                    acc_sh.at[pl.ds(s * WROWS, WROWS)])
    plsc.subcore_barrier()

    base = s * (CW0 + CW1) + c * CW0
    nslab = lax.select(c == 0, CW0 // SLAB, CW1 // SLAB)

    def gather(j, b):
        pltpu.async_copy(g_hbm.at[src_v.at[j]], rows[b], sems[b])

    def gwait(j, b):
        pltpu.make_async_copy(g_hbm.at[src_v.at[j]], rows[b],
                              sems[b]).wait()

    def slab_body(h, carry0):
        pltpu.sync_copy(srcp_hbm.at[pl.ds(base + h * SLAB, SLAB)], src_v)
        pltpu.sync_copy(dstp_hbm.at[pl.ds(base + h * SLAB, SLAB)], dst_v)
        for b in range(NBUF):
            gather(b, b)

        def body(j2, carry):
            for b in range(NBUF):
                j = j2 * NBUF + b
                gwait(j, b)
                pltpu.sync_copy(rows[b], acc_sh.at[dst_v.at[j]], add=True)

                @pl.when(j + NBUF < SLAB)
                def _():
                    gather(j + NBUF, b)
            return carry

        lax.fori_loop(0, SLAB // NBUF, body, 0)
        return carry0

    lax.fori_loop(0, nslab, slab_body, 0)
    plsc.subcore_barrier()
    pltpu.sync_copy(acc_sh.at[pl.ds(s * WROWS, WROWS)],
                    out_hbm.at[c, pl.ds(s * WROWS, WROWS)])


_agg_kernel = functools.partial(
    pl.kernel,
    out_type=jax.ShapeDtypeStruct((NC, NPAD, D), jnp.float32),
    mesh=_mesh,
    scratch_types=[
        pltpu.VMEM((SLAB, CHUNK), jnp.int32),
        pltpu.VMEM((SLAB, CHUNK), jnp.int32),
        pltpu.VMEM((CHUNK, D), jnp.float32),
        pltpu.VMEM((CHUNK, D), jnp.float32),
        pltpu.VMEM_SHARED((NPAD, D), jnp.float32),
        pltpu.SemaphoreType.DMA,
        pltpu.SemaphoreType.DMA,
    ],
)(_agg_body)


def _tcmm_body(x_ref, w_ref, h_ref):
    h_ref[...] = lax.dot_general(x_ref[...], w_ref[...],
                                 (((1,), (0,)), ((), ())),
                                 precision=lax.Precision.HIGHEST,
                                 preferred_element_type=jnp.float32)


def _tc_mm(x, W1):
    return pl.pallas_call(
        _tcmm_body,
        grid=(N // TCB,),
        in_specs=[
            pl.BlockSpec((TCB, D), lambda i: (i, 0)),
            pl.BlockSpec((D, D), lambda i: (0, 0)),
        ],
        out_specs=pl.BlockSpec((TCB, D), lambda i: (i, 0)),
        out_shape=jax.ShapeDtypeStruct((N, D), jnp.float32),
    )(x, W1)


def _tca_body(h_ref, dega_ref, degb_ref, g_ref, dinv_ref):
    deg = dega_ref[:, 0:1] + degb_ref[:, 0:1] + 1.0
    dinv = lax.rsqrt(deg)
    g_ref[...] = h_ref[...] * dinv
    dinv_ref[...] = jnp.broadcast_to(dinv, (TCB, 16))


def _tc_a(h, degA, degB):
    return pl.pallas_call(
        _tca_body,
        grid=(N // TCB,),
        in_specs=[
            pl.BlockSpec((TCB, D), lambda i: (i, 0)),
            pl.BlockSpec((TCB, 16), lambda i: (i, 0)),
            pl.BlockSpec((TCB, 16), lambda i: (i, 0)),
        ],
        out_specs=[
            pl.BlockSpec((TCB, D), lambda i: (i, 0)),
            pl.BlockSpec((TCB, 16), lambda i: (i, 0)),
        ],
        out_shape=[
            jax.ShapeDtypeStruct((N, D), jnp.float32),
            jax.ShapeDtypeStruct((N, 16), jnp.float32),
        ],
    )(h, degA, degB)


def _tcb_body(acca_ref, accb_ref, g1_ref, dinv_ref, b1_ref, w_ref, g2_ref):
    dv = dinv_ref[:, 0:1]
    pre = (acca_ref[...] + accb_ref[...] + g1_ref[...]) * dv + b1_ref[...]
    h1 = jnp.maximum(pre, 0.0)
    h2 = lax.dot_general(h1, w_ref[...], (((1,), (0,)), ((), ())),
                         precision=lax.Precision.HIGHEST,
                         preferred_element_type=jnp.float32)
    g2_ref[...] = h2 * dv


def _tc_b(accA, accB, g1, dinv, b1, W2):
    return pl.pallas_call(
        _tcb_body,
        grid=(N // TCB,),
        in_specs=[
            pl.BlockSpec((TCB, D), lambda i: (i, 0)),
            pl.BlockSpec((TCB, D), lambda i: (i, 0)),
            pl.BlockSpec((TCB, D), lambda i: (i, 0)),
            pl.BlockSpec((TCB, 16), lambda i: (i, 0)),
            pl.BlockSpec((1, D), lambda i: (0, 0)),
            pl.BlockSpec((D, D), lambda i: (0, 0)),
        ],
        out_specs=pl.BlockSpec((TCB, D), lambda i: (i, 0)),
        out_shape=jax.ShapeDtypeStruct((N, D), jnp.float32),
    )(accA, accB, g1, dinv, b1, W2)


def _tcc_body(acca_ref, accb_ref, g2_ref, dinv_ref, b2_ref, out_ref):
    dv = dinv_ref[:, 0:1]
    out_ref[...] = (acca_ref[...] + accb_ref[...] + g2_ref[...]) * dv \
        + b2_ref[...]


def _tc_c(accA, accB, g2, dinv, b2):
    return pl.pallas_call(
        _tcc_body,
        grid=(N // TCB,),
        in_specs=[
            pl.BlockSpec((TCB, D), lambda i: (i, 0)),
            pl.BlockSpec((TCB, D), lambda i: (i, 0)),
            pl.BlockSpec((TCB, D), lambda i: (i, 0)),
            pl.BlockSpec((TCB, 16), lambda i: (i, 0)),
            pl.BlockSpec((1, D), lambda i: (0, 0)),
        ],
        out_specs=pl.BlockSpec((TCB, D), lambda i: (i, 0)),
        out_shape=jax.ShapeDtypeStruct((N, D), jnp.float32),
    )(accA, accB, g2, dinv, b2)


@jax.jit
def kernel(x, edge_index, W1, b1, W2, b2):
    src = edge_index[0]
    dst = edge_index[1]
    pad = EPAD - E
    srcp = jnp.concatenate([src, jnp.zeros((pad,), jnp.int32)])
    dstp = jnp.concatenate([dst, jnp.full((pad,), N, jnp.int32)])
    srcp = srcp.reshape(NW * CW, CHUNK)
    dstp = dstp.reshape(NW * CW, CHUNK)

    onesD = jnp.ones((CHUNK, D), jnp.float32)
    zerosD = jnp.zeros((NPAD, D), jnp.float32)

    h1 = _tc_mm(x, W1)
    deg2 = _deg_kernel(dstp, onesD, zerosD)

    g1, dinv = _tc_a(h1, deg2[0, :N, :16], deg2[1, :N, :16])
    acc1 = _agg_kernel(g1, srcp, dstp, zerosD)
    g2 = _tc_b(acc1[0, :N], acc1[1, :N], g1, dinv, b1.reshape(1, D), W2)
    acc2 = _agg_kernel(g2, srcp, dstp, zerosD)
    out = _tc_c(acc2[0, :N], acc2[1, :N], g2, dinv, b2.reshape(1, D))
    return out


# speedup vs baseline: 1.0016x; 1.0016x over previous
import functools

import jax
import jax.numpy as jnp
from jax import lax
from jax.experimental import pallas as pl
from jax.experimental.pallas import tpu as pltpu
from jax.experimental.pallas import tpu_sc as plsc

N = 10000
E = 320000
D = 128

NC = 2
NS = 16
NW = NC * NS

CHUNK = 128
CW = 80
EPAD = NW * CW * CHUNK
NPAD = 10112
WROWS = NPAD // NS

TCB = 2000

_mesh = plsc.VectorSubcoreMesh(core_axis_name="c", subcore_axis_name="s")


def _deg_body(dstp_hbm, ones_hbm, zeros_hbm, out_hbm, dst_v, ones_v, acc_sh,
              sem):
    c = lax.axis_index("c")
    s = lax.axis_index("s")
    w = s * NC + c
    pltpu.sync_copy(zeros_hbm.at[pl.ds(s * WROWS, WROWS)],
                    acc_sh.at[pl.ds(s * WROWS, WROWS)])
    pltpu.sync_copy(ones_hbm, ones_v)
    pltpu.sync_copy(dstp_hbm.at[pl.ds(w * CW, CW)], dst_v)
    plsc.subcore_barrier()

    def body(j, carry):
        pltpu.sync_copy(ones_v, acc_sh.at[dst_v.at[j]], add=True)
        return carry

    lax.fori_loop(0, CW, body, 0)
    plsc.subcore_barrier()
    pltpu.sync_copy(acc_sh.at[pl.ds(s * WROWS, WROWS)],
                    out_hbm.at[c, pl.ds(s * WROWS, WROWS)])


_deg_kernel = functools.partial(
    pl.kernel,
    out_type=jax.ShapeDtypeStruct((NC, NPAD, D), jnp.float32),
    mesh=_mesh,
    scratch_types=[
        pltpu.VMEM((CW, CHUNK), jnp.int32),
        pltpu.VMEM((CHUNK, D), jnp.float32),
        pltpu.VMEM_SHARED((NPAD, D), jnp.float32),
        pltpu.SemaphoreType.DMA,
    ],
)(_deg_body)


NBUF = 2


CW0 = 120
CW1 = 2 * CW - CW0
SLAB = 40


def _agg_body(g_hbm, srcp_hbm, dstp_hbm, zeros_hbm, out_hbm, src_v, dst_v,
              r0, r1, acc_sh, *sems):
    c = lax.axis_index("c")
    s = lax.axis_index("s")
    rows = [r0, r1]
    pltpu.sync_copy(zeros_hbm.at[pl.ds(s * WROWS, WROWS)],
                    acc_sh.at[pl.ds(s * WROWS, WROWS)])
    plsc.subcore_barrier()

    base = s * (CW0 + CW1) + c * CW0
    nslab = lax.select(c == 0, CW0 // SLAB, CW1 // SLAB)

    def gather(j, b):
        pltpu.async_copy(g_hbm.at[src_v.at[j]], rows[b], sems[b])

    def gwait(j, b):
        pltpu.make_async_copy(g_hbm.at[src_v.at[j]], rows[b],
                              sems[b]).wait()

    def slab_body(h, carry0):
        pltpu.sync_copy(srcp_hbm.at[pl.ds(base + h * SLAB, SLAB)], src_v)
        pltpu.sync_copy(dstp_hbm.at[pl.ds(base + h * SLAB, SLAB)], dst_v)
        for b in range(NBUF):
            gather(b, b)

        def body(j2, carry):
            for b in range(NBUF):
                j = j2 * NBUF + b
                gwait(j, b)
                pltpu.sync_copy(rows[b], acc_sh.at[dst_v.at[j]], add=True)

                @pl.when(j + NBUF < SLAB)
                def _():
                    gather(j + NBUF, b)
            return carry

        lax.fori_loop(0, SLAB // NBUF, body, 0)
        return carry0

    lax.fori_loop(0, nslab, slab_body, 0)
    plsc.subcore_barrier()
    pltpu.sync_copy(acc_sh.at[pl.ds(s * WROWS, WROWS)],
                    out_hbm.at[c, pl.ds(s * WROWS, WROWS)])


_agg_kernel = functools.partial(
    pl.kernel,
    out_type=jax.ShapeDtypeStruct((NC, NPAD, D), jnp.float32),
    mesh=_mesh,
    scratch_types=[
        pltpu.VMEM((SLAB, CHUNK), jnp.int32),
        pltpu.VMEM((SLAB, CHUNK), jnp.int32),
        pltpu.VMEM((CHUNK, D), jnp.float32),
        pltpu.VMEM((CHUNK, D), jnp.float32),
        pltpu.VMEM_SHARED((NPAD, D), jnp.float32),
        pltpu.SemaphoreType.DMA,
        pltpu.SemaphoreType.DMA,
    ],
)(_agg_body)


def _tcmm_body(x_ref, w_ref, h_ref):
    h_ref[...] = lax.dot_general(x_ref[...], w_ref[...],
                                 (((1,), (0,)), ((), ())),
                                 precision=lax.Precision.HIGHEST,
                                 preferred_element_type=jnp.float32)


def _tc_mm(x, W1):
    return pl.pallas_call(
        _tcmm_body,
        grid=(N // TCB,),
        in_specs=[
            pl.BlockSpec((TCB, D), lambda i: (i, 0)),
            pl.BlockSpec((D, D), lambda i: (0, 0)),
        ],
        out_specs=pl.BlockSpec((TCB, D), lambda i: (i, 0)),
        out_shape=jax.ShapeDtypeStruct((N, D), jnp.float32),
    )(x, W1)


def _tca_body(h_ref, dega_ref, degb_ref, g_ref, dinv_ref):
    deg = dega_ref[:, 0:1] + degb_ref[:, 0:1] + 1.0
    dinv = lax.rsqrt(deg)
    g_ref[...] = h_ref[...] * dinv
    dinv_ref[...] = jnp.broadcast_to(dinv, (TCB, 16))


def _tc_a(h, degA, degB):
    return pl.pallas_call(
        _tca_body,
        grid=(N // TCB,),
        in_specs=[
            pl.BlockSpec((TCB, D), lambda i: (i, 0)),
            pl.BlockSpec((TCB, 16), lambda i: (i, 0)),
            pl.BlockSpec((TCB, 16), lambda i: (i, 0)),
        ],
        out_specs=[
            pl.BlockSpec((TCB, D), lambda i: (i, 0)),
            pl.BlockSpec((TCB, 16), lambda i: (i, 0)),
        ],
        out_shape=[
            jax.ShapeDtypeStruct((N, D), jnp.float32),
            jax.ShapeDtypeStruct((N, 16), jnp.float32),
        ],
    )(h, degA, degB)


def _tcb_body(acca_ref, accb_ref, g1_ref, dinv_ref, b1_ref, w_ref, g2_ref):
    dv = dinv_ref[:, 0:1]
    pre = (acca_ref[...] + accb_ref[...] + g1_ref[...]) * dv + b1_ref[...]
    h1 = jnp.maximum(pre, 0.0)
    h2 = lax.dot_general(h1, w_ref[...], (((1,), (0,)), ((), ())),
                         precision=lax.Precision.HIGHEST,
                         preferred_element_type=jnp.float32)
    g2_ref[...] = h2 * dv


def _tc_b(accA, accB, g1, dinv, b1, W2):
    return pl.pallas_call(
        _tcb_body,
        grid=(N // TCB,),
        in_specs=[
            pl.BlockSpec((TCB, D), lambda i: (i, 0)),
            pl.BlockSpec((TCB, D), lambda i: (i, 0)),
            pl.BlockSpec((TCB, D), lambda i: (i, 0)),
            pl.BlockSpec((TCB, 16), lambda i: (i, 0)),
            pl.BlockSpec((1, D), lambda i: (0, 0)),
            pl.BlockSpec((D, D), lambda i: (0, 0)),
        ],
        out_specs=pl.BlockSpec((TCB, D), lambda i: (i, 0)),
        out_shape=jax.ShapeDtypeStruct((N, D), jnp.float32),
    )(accA, accB, g1, dinv, b1, W2)


def _tcc_body(acca_ref, accb_ref, g2_ref, dinv_ref, b2_ref, out_ref):
    dv = dinv_ref[:, 0:1]
    out_ref[...] = (acca_ref[...] + accb_ref[...] + g2_ref[...]) * dv \
        + b2_ref[...]


def _tc_c(accA, accB, g2, dinv, b2):
    return pl.pallas_call(
        _tcc_body,
        grid=(N // TCB,),
        in_specs=[
            pl.BlockSpec((TCB, D), lambda i: (i, 0)),
            pl.BlockSpec((TCB, D), lambda i: (i, 0)),
            pl.BlockSpec((TCB, D), lambda i: (i, 0)),
            pl.BlockSpec((TCB, 16), lambda i: (i, 0)),
            pl.BlockSpec((1, D), lambda i: (0, 0)),
        ],
        out_specs=pl.BlockSpec((TCB, D), lambda i: (i, 0)),
        out_shape=jax.ShapeDtypeStruct((N, D), jnp.float32),
    )(accA, accB, g2, dinv, b2)


@jax.jit
def kernel(x, edge_index, W1, b1, W2, b2):
    src = edge_index[0]
    dst = edge_index[1]
    pad = EPAD - E
    srcp = jnp.concatenate([src, jnp.zeros((pad,), jnp.int32)])
    dstp = jnp.concatenate([dst, jnp.full((pad,), N, jnp.int32)])
    srcp = srcp.reshape(NW * CW, CHUNK)
    dstp = dstp.reshape(NW * CW, CHUNK)

    onesD = jnp.ones((CHUNK, D), jnp.float32)
    zerosD = jnp.zeros((NPAD, D), jnp.float32)

    h1 = _tc_mm(x, W1)
    deg2 = _deg_kernel(dstp, onesD, zerosD)

    g1, dinv = _tc_a(h1, deg2[0, :N, :16], deg2[1, :N, :16])
    acc1 = _agg_kernel(g1, srcp, dstp, zerosD)
    g2 = _tc_b(acc1[0, :N], acc1[1, :N], g1, dinv, b1.reshape(1, D), W2)
    acc2 = _agg_kernel(g2, srcp, dstp, zerosD)
    out = _tc_c(acc2[0, :N], acc2[1, :N], g2, dinv, b2.reshape(1, D))
    return out
